# Initial kernel scaffold; baseline (speedup 1.0000x reference)
import jax
import jax.numpy as jnp
from jax.experimental import pallas as pl


def kernel(mol1_x, mol1_edge_index, mol1_edge_attr, mol1_batch, mol2_x, mol2_edge_index, mol2_edge_attr, mol2_batch, params):
    raise NotImplementedError("write your pallas kernel here")


# speedup vs baseline: 2.1323x; 2.1323x over previous
import functools

import jax
import jax.numpy as jnp
from jax import lax
from jax.experimental import pallas as pl
from jax.experimental.pallas import tpu as pltpu
from jax.experimental.pallas import tpu_sc as plsc

N = 10000
E = 320000
D = 128
DE = 16
NB = 128
NT = 79
NP = NT * 128
RPT = NP // 16
EPT = E // 16
EC = 80
NEG = -jnp.inf
BIGF = 1e9
BIGNEG = 1e30

def _sc_mesh():
    return plsc.VectorSubcoreMesh(core_axis_name="c", subcore_axis_name="s")


def _segsum_gather(table, srcg, dstg, zeros_np):
    @functools.partial(
        pl.kernel,
        mesh=_sc_mesh(),
        out_type=jax.ShapeDtypeStruct((2 * NP, D), jnp.float32),
        scratch_types=[
            pltpu.VMEM((EC,), jnp.int32),
            pltpu.VMEM((EC,), jnp.int32),
            pltpu.VMEM((EC, D), jnp.float32),
            pltpu.SemaphoreType.DMA,
            pltpu.VMEM_SHARED((NP, D), jnp.float32),
        ],
    )
    def k(table_h, src_h, dst_h, zeros_h, out_h, src_v, dst_v, rows_v, sem, accum):
        cid = lax.axis_index("c")
        sid = lax.axis_index("s")
        pltpu.sync_copy(zeros_h.at[pl.ds(sid * RPT, RPT)],
                        accum.at[pl.ds(sid * RPT, RPT)])
        plsc.subcore_barrier()
        ebase = cid * E + sid * EPT

        def chunk(i, carry):
            off = ebase + i * EC
            pltpu.sync_copy(src_h.at[pl.ds(off, EC)], src_v)
            pltpu.sync_copy(dst_h.at[pl.ds(off, EC)], dst_v)
            pltpu.async_copy(table_h.at[src_v], rows_v, sem).wait()
            pltpu.sync_copy(rows_v, accum.at[dst_v], add=True)
            return carry

        lax.fori_loop(0, EPT // EC, chunk, 0)
        plsc.subcore_barrier()
        pltpu.sync_copy(accum.at[pl.ds(sid * RPT, RPT)],
                        out_h.at[pl.ds(cid * NP + sid * RPT, RPT)])

    return k(table, srcg, dstg, zeros_np)


def _segsum_linear(table, dstg, zeros_np):
    @functools.partial(
        pl.kernel,
        mesh=_sc_mesh(),
        out_type=jax.ShapeDtypeStruct((2 * NP, D), jnp.float32),
        scratch_types=[
            pltpu.VMEM((EC,), jnp.int32),
            pltpu.VMEM((EC, D), jnp.float32),
            pltpu.SemaphoreType.DMA,
            pltpu.VMEM_SHARED((NP, D), jnp.float32),
        ],
    )
    def k(table_h, dst_h, zeros_h, out_h, dst_v, rows_v, sem, accum):
        cid = lax.axis_index("c")
        sid = lax.axis_index("s")
        pltpu.sync_copy(zeros_h.at[pl.ds(sid * RPT, RPT)],
                        accum.at[pl.ds(sid * RPT, RPT)])
        plsc.subcore_barrier()
        ebase = cid * E + sid * EPT

        def chunk(i, carry):
            off = ebase + i * EC
            pltpu.sync_copy(dst_h.at[pl.ds(off, EC)], dst_v)
            pltpu.sync_copy(table_h.at[pl.ds(off, EC)], rows_v)
            pltpu.sync_copy(rows_v, accum.at[dst_v], add=True)
            return carry

        lax.fori_loop(0, EPT // EC, chunk, 0)
        plsc.subcore_barrier()
        pltpu.sync_copy(accum.at[pl.ds(sid * RPT, RPT)],
                        out_h.at[pl.ds(cid * NP + sid * RPT, RPT)])

    return k(table, dstg, zeros_np)


def _leaky(v):
    return jnp.where(v >= 0, v, 0.22916667 * v)


def _init_body(x_ref, w_ref, b_ref, o_ref):
    o_ref[0] = _leaky(
        jnp.dot(x_ref[0], w_ref[0], preferred_element_type=jnp.float32, precision=lax.Precision.HIGHEST)
        + b_ref[0])


def _init_x(xs, w0, b0):
    return pl.pallas_call(
        _init_body,
        grid=(2, NT),
        in_specs=[
            pl.BlockSpec((1, 128, D), lambda m, t: (m, t, 0)),
            pl.BlockSpec((1, D, D), lambda m, t: (m, 0, 0)),
            pl.BlockSpec((1, 1, D), lambda m, t: (m, 0, 0)),
        ],
        out_specs=pl.BlockSpec((1, 128, D), lambda m, t: (m, t, 0)),
        out_shape=jax.ShapeDtypeStruct((2, NP, D), jnp.float32),
    )(xs, w0, b0)


def _edge_body(ea_ref, w_ref, b_ref, o_ref):
    o_ref[0] = jnp.maximum(
        jnp.dot(ea_ref[0], w_ref[0], preferred_element_type=jnp.float32, precision=lax.Precision.HIGHEST)
        + b_ref[0], 0.0)


def _edge_feat(eas, we, be):
    ET = 2000
    return pl.pallas_call(
        _edge_body,
        grid=(2, E // ET),
        in_specs=[
            pl.BlockSpec((1, ET, DE), lambda m, t: (m, t, 0)),
            pl.BlockSpec((1, DE, D), lambda m, t: (m, 0, 0)),
            pl.BlockSpec((1, 1, D), lambda m, t: (m, 0, 0)),
        ],
        out_specs=pl.BlockSpec((1, ET, D), lambda m, t: (m, t, 0)),
        out_shape=jax.ShapeDtypeStruct((2, E, D), jnp.float32),
    )(eas, we, be)


def _gru_body(sx_ref, se_ref, h_ref, wm_ref, wzr_ref, uzr_ref, bzr_ref,
              wh_ref, uh_ref, bh_ref, o_ref):
    f32 = jnp.float32
    a = jnp.dot(sx_ref[0] + se_ref[0], wm_ref[0], preferred_element_type=f32, precision=lax.Precision.HIGHEST)
    h = h_ref[0]
    zr = jax.nn.sigmoid(
        jnp.dot(a, wzr_ref[0], preferred_element_type=f32, precision=lax.Precision.HIGHEST)
        + jnp.dot(h, uzr_ref[0], preferred_element_type=f32, precision=lax.Precision.HIGHEST) + bzr_ref[0])
    z = zr[:, :D]
    r = zr[:, D:]
    n = jnp.tanh(
        jnp.dot(a, wh_ref[0], preferred_element_type=f32, precision=lax.Precision.HIGHEST)
        + jnp.dot(r * h, uh_ref[0], preferred_element_type=f32, precision=lax.Precision.HIGHEST) + bh_ref[0])
    o_ref[0] = (1.0 - z) * n + z * h


def _gru(sx, se, h, wm, wzr, uzr, bzr, wh, uh, bh):
    return pl.pallas_call(
        _gru_body,
        grid=(2, NT),
        in_specs=[
            pl.BlockSpec((1, 128, D), lambda m, t: (m, t, 0)),
            pl.BlockSpec((1, 128, D), lambda m, t: (m, t, 0)),
            pl.BlockSpec((1, 128, D), lambda m, t: (m, t, 0)),
            pl.BlockSpec((1, D, D), lambda m, t: (m, 0, 0)),
            pl.BlockSpec((1, D, 2 * D), lambda m, t: (m, 0, 0)),
            pl.BlockSpec((1, D, 2 * D), lambda m, t: (m, 0, 0)),
            pl.BlockSpec((1, 1, 2 * D), lambda m, t: (m, 0, 0)),
            pl.BlockSpec((1, D, D), lambda m, t: (m, 0, 0)),
            pl.BlockSpec((1, D, D), lambda m, t: (m, 0, 0)),
            pl.BlockSpec((1, 1, D), lambda m, t: (m, 0, 0)),
        ],
        out_specs=pl.BlockSpec((1, 128, D), lambda m, t: (m, t, 0)),
        out_shape=jax.ShapeDtypeStruct((2, NP, D), jnp.float32),
    )(sx, se, h, wm, wzr, uzr, bzr, wh, uh, bh)


def _meta_body(b1r_ref, b2r_ref, c1_ref, c2_ref, s2i_ref, c2i_ref, acc1, acc2):
    t = pl.program_id(0)
    iota_b = lax.broadcasted_iota(jnp.int32, (NB, 128), 0).astype(jnp.float32)

    @pl.when(t == 0)
    def _():
        acc1[...] = jnp.zeros((NB, 1), jnp.float32)
        acc2[...] = jnp.zeros((NB, 1), jnp.float32)

    oh1 = (iota_b == b1r_ref[0]).astype(jnp.float32)
    oh2 = (iota_b == b2r_ref[0]).astype(jnp.float32)
    ones = jnp.ones((128, 1), jnp.float32)
    acc1[...] += jnp.dot(oh1, ones, preferred_element_type=jnp.float32, precision=lax.Precision.HIGHEST)
    acc2[...] += jnp.dot(oh2, ones, preferred_element_type=jnp.float32, precision=lax.Precision.HIGHEST)

    @pl.when(t == NT - 1)
    def _():
        c1_ref[...] = acc1[...]
        c2_ref[...] = acc2[...]
        lt = (lax.broadcasted_iota(jnp.int32, (NB, NB), 1)
              < lax.broadcasted_iota(jnp.int32, (NB, NB), 0)
              ).astype(jnp.float32)
        s2 = jnp.dot(lt, acc2[...], preferred_element_type=jnp.float32, precision=lax.Precision.HIGHEST)
        s2i_ref[...] = s2.astype(jnp.int32)
        c2i_ref[...] = acc2[...].astype(jnp.int32)


def _meta(b1r, b2r):
    return pl.pallas_call(
        _meta_body,
        grid=(NT,),
        in_specs=[
            pl.BlockSpec((1, 1, 128), lambda t: (t, 0, 0)),
            pl.BlockSpec((1, 1, 128), lambda t: (t, 0, 0)),
        ],
        out_specs=[
            pl.BlockSpec((NB, 1), lambda t: (0, 0)),
            pl.BlockSpec((NB, 1), lambda t: (0, 0)),
            pl.BlockSpec((NB, 1), lambda t: (0, 0)),
            pl.BlockSpec((NB, 1), lambda t: (0, 0)),
        ],
        out_shape=[
            jax.ShapeDtypeStruct((NB, 1), jnp.float32),
            jax.ShapeDtypeStruct((NB, 1), jnp.float32),
            jax.ShapeDtypeStruct((NB, 1), jnp.int32),
            jax.ShapeDtypeStruct((NB, 1), jnp.int32),
        ],
        scratch_shapes=[
            pltpu.VMEM((NB, 1), jnp.float32),
            pltpu.VMEM((NB, 1), jnp.float32),
        ],
    )(b1r, b2r)


def _eye128():
    return (lax.broadcasted_iota(jnp.int32, (128, 128), 0)
            == lax.broadcasted_iota(jnp.int32, (128, 128), 1)
            ).astype(jnp.float32)


def _pool_body(x1_ref, x2f_ref, b1r_ref, b2r_ref, b2f_ref,
               c1_ref, c2_ref, s2i_ref, c2i_ref, tb_ref,
               t1_ref, t2_ref, fus_ref, smax_acc):
    f32 = jnp.float32
    i = pl.program_id(0)

    @pl.when(i == 0)
    def _():
        t1_ref[...] = jnp.zeros((NB, D), f32)
        t2_ref[...] = jnp.zeros((NB, D), f32)
        smax_acc[...] = jnp.full((NB, 1), -BIGNEG, f32)

    x1t = x1_ref[0]
    b1row = b1r_ref[0]
    b1col = lax.dot_general(_eye128(), b1row, (((1,), (1,)), ((), ())),
                            preferred_element_type=f32, precision=lax.Precision.HIGHEST)
    iota_col = lax.broadcasted_iota(jnp.int32, (NB, 128), 0).astype(f32)
    oh1 = (iota_col == b1row).astype(f32)
    oh2 = (iota_col == b2r_ref[0]).astype(f32)
    t1_ref[...] += jnp.dot(oh1, x1t, preferred_element_type=f32, precision=lax.Precision.HIGHEST)
    t2_ref[...] += jnp.dot(oh2, x2f_ref[0, pl.ds(i * 128, 128), :],
                           preferred_element_type=f32, precision=lax.Precision.HIGHEST)

    bmin = jnp.minimum(tb_ref[i, 0], NB - 1)
    bmax = jnp.minimum(tb_ref[i, 1], NB - 1)
    cs = s2i_ref[bmin, 0]
    ce = s2i_ref[bmax, 0] + c2i_ref[bmax, 0]
    jlo = lax.div(cs, 128)
    jhi = lax.div(ce + 127, 128)

    def col_step(j, rmax):
        x2t = x2f_ref[0, pl.ds(j * 128, 128), :]
        s = lax.dot_general(x1t, x2t, (((1,), (1,)), ((), ())),
                            preferred_element_type=f32, precision=lax.Precision.HIGHEST)
        b2row = b2f_ref[pl.ds(j, 1), :]
        m = jnp.where(b1col == b2row, s, -BIGNEG)
        return jnp.maximum(rmax, jnp.max(m, axis=1, keepdims=True))

    rmax = lax.fori_loop(jlo, jhi, col_step,
                         jnp.full((128, 1), -BIGNEG, f32))
    rmax_row = lax.dot_general(rmax, _eye128(), (((0,), (0,)), ((), ())),
                               preferred_element_type=f32, precision=lax.Precision.HIGHEST)
    contrib = jnp.max(jnp.where(iota_col == b1row, rmax_row, -BIGNEG),
                      axis=1, keepdims=True)
    smax_acc[...] = jnp.maximum(smax_acc[...], contrib)

    @pl.when(i == NT - 1)
    def _():
        ssum = jnp.sum(t1_ref[...] * t2_ref[...], axis=1, keepdims=True)
        mean = ssum / (c1_ref[...] * c2_ref[...])
        fus_ref[...] = jnp.concatenate([smax_acc[...], mean], axis=1)


def _pool(x, b1r3, b2r3, b2r, c1f, c2f, s2i, c2i, tb):
    return pl.pallas_call(
        _pool_body,
        grid=(NT,),
        in_specs=[
            pl.BlockSpec((1, 128, D), lambda t: (0, t, 0)),
            pl.BlockSpec((1, NP, D), lambda t: (1, 0, 0)),
            pl.BlockSpec((1, 1, 128), lambda t: (t, 0, 0)),
            pl.BlockSpec((1, 1, 128), lambda t: (t, 0, 0)),
            pl.BlockSpec((NT, 128), lambda t: (0, 0)),
            pl.BlockSpec((NB, 1), lambda t: (0, 0)),
            pl.BlockSpec((NB, 1), lambda t: (0, 0)),
            pl.BlockSpec(memory_space=pltpu.SMEM),
            pl.BlockSpec(memory_space=pltpu.SMEM),
            pl.BlockSpec(memory_space=pltpu.SMEM),
        ],
        out_specs=[
            pl.BlockSpec((NB, D), lambda t: (0, 0)),
            pl.BlockSpec((NB, D), lambda t: (0, 0)),
            pl.BlockSpec((NB, 2), lambda t: (0, 0)),
        ],
        out_shape=[
            jax.ShapeDtypeStruct((NB, D), jnp.float32),
            jax.ShapeDtypeStruct((NB, D), jnp.float32),
            jax.ShapeDtypeStruct((NB, 2), jnp.float32),
        ],
        scratch_shapes=[pltpu.VMEM((NB, 1), jnp.float32)],
    )(x, x, b1r3, b2r3, b2r, c1f, c2f, s2i, c2i, tb)


def _head_body(x_ref, b1r_ref, b2r_ref, c1_ref, c2_ref, t1_ref, t2_ref,
               fus_ref, wf_ref, bf_ref, wo1_ref, bo1_ref, wo2_ref, bo2_ref,
               o_ref):
    f32 = jnp.float32
    iota_col = lax.broadcasted_iota(jnp.int32, (NB, 128), 0).astype(f32)
    eye = _eye128()
    os = []
    for m in range(2):
        br_ref = b1r_ref if m == 0 else b2r_ref
        cnt = c1_ref[...] if m == 0 else c2_ref[...]
        t = t1_ref[...] if m == 0 else t2_ref[...]

        def keyrow(tt):
            kc = x_ref[m, pl.ds(tt * 128, 128), 127:128]
            return lax.dot_general(kc, eye, (((0,), (0,)), ((), ())),
                                   preferred_element_type=f32, precision=lax.Precision.HIGHEST)

        def tile_scan(excl0, excl1):
            def step(tt, carry):
                v, ix = carry
                krow = keyrow(tt)
                oh = iota_col == br_ref[pl.ds(tt, 1), :]
                gcol = (lax.broadcasted_iota(jnp.int32, (NB, 128), 1).astype(f32)
                        + 128.0 * tt.astype(f32))
                mm = jnp.where(oh, jnp.broadcast_to(krow, (NB, 128)), NEG)
                if excl0 is not None:
                    mm = jnp.where(gcol == excl0, NEG, mm)
                if excl1 is not None:
                    mm = jnp.where(gcol == excl1, NEG, mm)
                tv = jnp.max(mm, axis=1, keepdims=True)
                tix = jnp.min(jnp.where(mm == tv, gcol, BIGF),
                              axis=1, keepdims=True)
                better = tv > v
                return (jnp.where(better, tv, v),
                        jnp.where(better, tix, ix))

            init = (jnp.full((NB, 1), NEG, f32), jnp.full((NB, 1), BIGF, f32))
            return lax.fori_loop(0, NT, step, init)

        _, i0 = tile_scan(None, None)
        _, i1 = tile_scan(i0, None)
        _, i2 = tile_scan(i0, i1)

        def gather(tt, accs):
            a0, a1, a2 = accs
            xt = x_ref[m, pl.ds(tt * 128, 128), :]
            gcol = (lax.broadcasted_iota(jnp.int32, (NB, 128), 1).astype(f32)
                    + 128.0 * tt.astype(f32))
            a0 += jnp.dot((gcol == i0).astype(f32), xt,
                          preferred_element_type=f32, precision=lax.Precision.HIGHEST)
            a1 += jnp.dot((gcol == i1).astype(f32), xt,
                          preferred_element_type=f32, precision=lax.Precision.HIGHEST)
            a2 += jnp.dot((gcol == i2).astype(f32), xt,
                          preferred_element_type=f32, precision=lax.Precision.HIGHEST)
            return a0, a1, a2

        z = jnp.zeros((NB, D), f32)
        a0, a1, a2 = lax.fori_loop(0, NT, gather, (z, z, z))
        a0 = jnp.where(cnt > 0.0, a0, 0.0)
        a1 = jnp.where(cnt > 1.0, a1, 0.0)
        a2 = jnp.where(cnt > 2.0, a2, 0.0)
        mean = t / jnp.maximum(cnt, 1.0)
        gp = jnp.concatenate([mean, t, a0, a1, a2], axis=1)
        os.append(jnp.dot(gp, wf_ref[m], preferred_element_type=f32, precision=lax.Precision.HIGHEST)
                  + bf_ref[m])
    cat = jnp.concatenate(
        [os[0], os[1], fus_ref[0], fus_ref[1], fus_ref[2]], axis=1)
    h1 = jnp.dot(cat, wo1_ref[...], preferred_element_type=f32, precision=lax.Precision.HIGHEST) + bo1_ref[...]
    o_ref[...] = (jnp.dot(h1, wo2_ref[...], preferred_element_type=f32, precision=lax.Precision.HIGHEST)
                  + bo2_ref[...])


def _head(x, b1r, b2r, c1f, c2f, t1, t2, fus, wf, bf, wo1, bo1, wo2, bo2):
    return pl.pallas_call(
        _head_body,
        in_specs=[
            pl.BlockSpec((2, NP, D), lambda: (0, 0, 0)),
            pl.BlockSpec((NT, 128), lambda: (0, 0)),
            pl.BlockSpec((NT, 128), lambda: (0, 0)),
            pl.BlockSpec((NB, 1), lambda: (0, 0)),
            pl.BlockSpec((NB, 1), lambda: (0, 0)),
            pl.BlockSpec((NB, D), lambda: (0, 0)),
            pl.BlockSpec((NB, D), lambda: (0, 0)),
            pl.BlockSpec((3, NB, 2), lambda: (0, 0, 0)),
            pl.BlockSpec((2, 5 * D, D), lambda: (0, 0, 0)),
            pl.BlockSpec((2, 1, D), lambda: (0, 0, 0)),
            pl.BlockSpec((2 * D + 6, D), lambda: (0, 0)),
            pl.BlockSpec((1, D), lambda: (0, 0)),
            pl.BlockSpec((D, 2), lambda: (0, 0)),
            pl.BlockSpec((1, 2), lambda: (0, 0)),
        ],
        out_specs=pl.BlockSpec((NB, 2), lambda: (0, 0)),
        out_shape=jax.ShapeDtypeStruct((NB, 2), jnp.float32),
    )(x, b1r, b2r, c1f, c2f, t1, t2, fus, wf, bf, wo1, bo1, wo2, bo2)


def kernel(mol1_x, mol1_edge_index, mol1_edge_attr, mol1_batch,
           mol2_x, mol2_edge_index, mol2_edge_attr, mol2_batch, params):
    p = params
    f32 = jnp.float32

    xs = jnp.zeros((2, NP, D), f32)
    xs = xs.at[0, :N].set(mol1_x).at[1, :N].set(mol2_x)
    eas = jnp.stack([mol1_edge_attr, mol2_edge_attr])
    b1p = jnp.concatenate([mol1_batch,
                           jnp.full((NP - N,), NB, jnp.int32)])
    b2p = jnp.concatenate([mol2_batch,
                           jnp.full((NP - N,), NB, jnp.int32)])
    b1r = b1p.reshape(NT, 128).astype(f32)
    b2r = b2p.reshape(NT, 128).astype(f32)
    b1r3 = b1r.reshape(NT, 1, 128)
    b2r3 = b2r.reshape(NT, 1, 128)
    tb = jnp.stack([b1p[::128], b1p[127::128]], axis=1)
    srcg = jnp.concatenate([mol1_edge_index[0],
                            mol2_edge_index[0] + NP])
    dstg = jnp.concatenate([mol1_edge_index[1],
                            mol2_edge_index[1]])
    zeros_np = jnp.zeros((NP, D), f32)

    def stack2(*names):
        return [jnp.stack([p['m1_' + nm], p['m2_' + nm]]) for nm in names]

    w0, we, wm, wh, uh = stack2('W0', 'We', 'Wm', 'Wh', 'Uh')
    b0, be, bh = [b.reshape(2, 1, D) for b in stack2('b0', 'be', 'bh')]
    wzr = jnp.concatenate(
        [jnp.stack([p['m1_Wz'], p['m2_Wz']]),
         jnp.stack([p['m1_Wr'], p['m2_Wr']])], axis=2)
    uzr = jnp.concatenate(
        [jnp.stack([p['m1_Uz'], p['m2_Uz']]),
         jnp.stack([p['m1_Ur'], p['m2_Ur']])], axis=2)
    bzr = jnp.concatenate(
        [jnp.stack([p['m1_bz'], p['m2_bz']]),
         jnp.stack([p['m1_br'], p['m2_br']])], axis=1).reshape(2, 1, 2 * D)
    wf = jnp.stack([p['m1_Wf'], p['m2_Wf']])
    bf = jnp.stack([p['m1_bf'], p['m2_bf']]).reshape(2, 1, D)

    x = _init_x(xs, w0, b0)
    y = _edge_feat(eas, we, be).reshape(2 * E, D)
    se = _segsum_linear(y, dstg, zeros_np).reshape(2, NP, D)
    c1f, c2f, s2i, c2i = _meta(b1r3, b2r3)

    fus = []
    t1 = t2 = None
    for _ in range(3):
        sx = _segsum_gather(x.reshape(2 * NP, D), srcg, dstg,
                            zeros_np).reshape(2, NP, D)
        x = _gru(sx, se, x, wm, wzr, uzr, bzr, wh, uh, bh)
        t1, t2, f = _pool(x, b1r3, b2r3, b2r, c1f, c2f, s2i, c2i, tb)
        fus.append(f)

    fus3 = jnp.stack(fus)
    return _head(x, b1r, b2r, c1f, c2f, t1, t2, fus3, wf, bf,
                 p['Wo1'], p['bo1'].reshape(1, D),
                 p['Wo2'], p['bo2'].reshape(1, 2))


# speedup vs baseline: 2.9428x; 1.3801x over previous
import functools

import jax
import jax.numpy as jnp
from jax import lax
from jax.experimental import pallas as pl
from jax.experimental.pallas import tpu as pltpu
from jax.experimental.pallas import tpu_sc as plsc

N = 10000
E = 320000
D = 128
DE = 16
NB = 128
NT = 79
NP = NT * 128
RPT = NP // 16
EPT = E // 16
EC = 80
NEG = -jnp.inf
BIGF = 1e9
BIGNEG = 1e30

def _sc_mesh():
    return plsc.VectorSubcoreMesh(core_axis_name="c", subcore_axis_name="s")


def _segsum_gather(table, srcg, dstg, zeros_np):
    NCH = EPT // EC

    @functools.partial(
        pl.kernel,
        mesh=_sc_mesh(),
        out_type=jax.ShapeDtypeStruct((2 * NP, D), jnp.float32),
        scratch_types=[
            pltpu.VMEM((EC,), jnp.int32),
            pltpu.VMEM((EC,), jnp.int32),
            pltpu.VMEM((EC,), jnp.int32),
            pltpu.VMEM((EC,), jnp.int32),
            pltpu.VMEM((EC, D), jnp.float32),
            pltpu.VMEM((EC, D), jnp.float32),
            pltpu.SemaphoreType.DMA,
            pltpu.SemaphoreType.DMA,
            pltpu.SemaphoreType.DMA,
            pltpu.SemaphoreType.DMA,
            pltpu.SemaphoreType.DMA,
            pltpu.SemaphoreType.DMA,
            pltpu.VMEM_SHARED((NP, D), jnp.float32),
        ],
    )
    def k(table_h, src_h, dst_h, zeros_h, out_h,
          src_a, src_b, dst_a, dst_b, rows_a, rows_b,
          sa, sb, da, db, sem_a, sem_b, accum):
        cid = lax.axis_index("c")
        sid = lax.axis_index("s")
        pltpu.sync_copy(zeros_h.at[pl.ds(sid * RPT, RPT)],
                        accum.at[pl.ds(sid * RPT, RPT)])
        plsc.subcore_barrier()
        ebase = cid * E + sid * EPT

        def idxload(i, srcv, dstv, ssem, dsem):
            off = ebase + i * EC
            pltpu.async_copy(src_h.at[pl.ds(off, EC)], srcv, ssem)
            pltpu.async_copy(dst_h.at[pl.ds(off, EC)], dstv, dsem)

        def wait_idx(srcv, dstv, ssem, dsem):
            pltpu.make_async_copy(src_h.at[pl.ds(ebase, EC)], srcv, ssem).wait()
            pltpu.make_async_copy(dst_h.at[pl.ds(ebase, EC)], dstv, dsem).wait()

        def gath(srcv, rows, sem):
            pltpu.async_copy(table_h.at[srcv], rows, sem)

        def wait_rows(rows, sem):
            pltpu.make_async_copy(table_h.at[pl.ds(0, EC)], rows, sem).wait()

        def scat(rows, dstv):
            pltpu.sync_copy(rows, accum.at[dstv], add=True)

        idxload(0, src_a, dst_a, sa, da)
        wait_idx(src_a, dst_a, sa, da)
        gath(src_a, rows_a, sem_a)
        idxload(1, src_b, dst_b, sb, db)

        def pair(kk, carry):
            i0 = 2 * kk
            pltpu.make_async_copy(src_h.at[pl.ds(ebase, EC)], src_b, sb).wait()
            pltpu.make_async_copy(dst_h.at[pl.ds(ebase, EC)], dst_b, db).wait()
            gath(src_b, rows_b, sem_b)
            wait_rows(rows_a, sem_a)
            scat(rows_a, dst_a)
            idxload(i0 + 2, src_a, dst_a, sa, da)
            wait_idx(src_a, dst_a, sa, da)
            gath(src_a, rows_a, sem_a)
            wait_rows(rows_b, sem_b)
            scat(rows_b, dst_b)
            idxload(i0 + 3, src_b, dst_b, sb, db)
            return carry

        lax.fori_loop(0, NCH // 2 - 1, pair, 0)
        pltpu.make_async_copy(src_h.at[pl.ds(ebase, EC)], src_b, sb).wait()
        pltpu.make_async_copy(dst_h.at[pl.ds(ebase, EC)], dst_b, db).wait()
        gath(src_b, rows_b, sem_b)
        wait_rows(rows_a, sem_a)
        scat(rows_a, dst_a)
        wait_rows(rows_b, sem_b)
        scat(rows_b, dst_b)

        plsc.subcore_barrier()
        pltpu.sync_copy(accum.at[pl.ds(sid * RPT, RPT)],
                        out_h.at[pl.ds(cid * NP + sid * RPT, RPT)])

    return k(table, srcg, dstg, zeros_np)


def _segsum_linear(table, dstg, zeros_np):
    NCH = EPT // EC

    @functools.partial(
        pl.kernel,
        mesh=_sc_mesh(),
        out_type=jax.ShapeDtypeStruct((2 * NP, D), jnp.float32),
        scratch_types=[
            pltpu.VMEM((EC,), jnp.int32),
            pltpu.VMEM((EC,), jnp.int32),
            pltpu.VMEM((EC, D), jnp.float32),
            pltpu.VMEM((EC, D), jnp.float32),
            pltpu.SemaphoreType.DMA,
            pltpu.SemaphoreType.DMA,
            pltpu.SemaphoreType.DMA,
            pltpu.SemaphoreType.DMA,
            pltpu.VMEM_SHARED((NP, D), jnp.float32),
        ],
    )
    def k(table_h, dst_h, zeros_h, out_h, dst_a, dst_b, rows_a, rows_b,
          da, db, sem_a, sem_b, accum):
        cid = lax.axis_index("c")
        sid = lax.axis_index("s")
        pltpu.sync_copy(zeros_h.at[pl.ds(sid * RPT, RPT)],
                        accum.at[pl.ds(sid * RPT, RPT)])
        plsc.subcore_barrier()
        ebase = cid * E + sid * EPT

        def idxload(i, dstv, dsem):
            pltpu.async_copy(dst_h.at[pl.ds(ebase + i * EC, EC)], dstv, dsem)

        def wait_idx(dstv, dsem):
            pltpu.make_async_copy(dst_h.at[pl.ds(ebase, EC)], dstv,
                                  dsem).wait()

        def gath(i, rows, sem):
            pltpu.async_copy(table_h.at[pl.ds(ebase + i * EC, EC)], rows, sem)

        def wait_rows(rows, sem):
            pltpu.make_async_copy(table_h.at[pl.ds(0, EC)], rows, sem).wait()

        def scat(rows, dstv):
            pltpu.sync_copy(rows, accum.at[dstv], add=True)

        idxload(0, dst_a, da)
        gath(0, rows_a, sem_a)
        idxload(1, dst_b, db)

        def pair(kk, carry):
            i0 = 2 * kk
            gath(i0 + 1, rows_b, sem_b)
            wait_rows(rows_a, sem_a)
            wait_idx(dst_a, da)
            scat(rows_a, dst_a)
            idxload(i0 + 2, dst_a, da)
            gath(i0 + 2, rows_a, sem_a)
            wait_rows(rows_b, sem_b)
            wait_idx(dst_b, db)
            scat(rows_b, dst_b)
            idxload(i0 + 3, dst_b, db)
            return carry

        lax.fori_loop(0, NCH // 2 - 1, pair, 0)
        gath(NCH - 1, rows_b, sem_b)
        wait_rows(rows_a, sem_a)
        wait_idx(dst_a, da)
        scat(rows_a, dst_a)
        wait_rows(rows_b, sem_b)
        wait_idx(dst_b, db)
        scat(rows_b, dst_b)

        plsc.subcore_barrier()
        pltpu.sync_copy(accum.at[pl.ds(sid * RPT, RPT)],
                        out_h.at[pl.ds(cid * NP + sid * RPT, RPT)])

    return k(table, dstg, zeros_np)


def _leaky(v):
    return jnp.where(v >= 0, v, 0.22916667 * v)


def _init_body(x_ref, w_ref, b_ref, o_ref):
    o_ref[0] = _leaky(
        jnp.dot(x_ref[0], w_ref[0], preferred_element_type=jnp.float32, precision=lax.Precision.HIGHEST)
        + b_ref[0])


def _init_x(xs, w0, b0):
    return pl.pallas_call(
        _init_body,
        grid=(2, NT),
        in_specs=[
            pl.BlockSpec((1, 128, D), lambda m, t: (m, t, 0)),
            pl.BlockSpec((1, D, D), lambda m, t: (m, 0, 0)),
            pl.BlockSpec((1, 1, D), lambda m, t: (m, 0, 0)),
        ],
        out_specs=pl.BlockSpec((1, 128, D), lambda m, t: (m, t, 0)),
        out_shape=jax.ShapeDtypeStruct((2, NP, D), jnp.float32),
    )(xs, w0, b0)


def _edge_body(ea_ref, w_ref, b_ref, o_ref):
    o_ref[0] = jnp.maximum(
        jnp.dot(ea_ref[0], w_ref[0], preferred_element_type=jnp.float32, precision=lax.Precision.HIGHEST)
        + b_ref[0], 0.0)


def _edge_feat(eas, we, be):
    ET = 2000
    return pl.pallas_call(
        _edge_body,
        grid=(2, E // ET),
        in_specs=[
            pl.BlockSpec((1, ET, DE), lambda m, t: (m, t, 0)),
            pl.BlockSpec((1, DE, D), lambda m, t: (m, 0, 0)),
            pl.BlockSpec((1, 1, D), lambda m, t: (m, 0, 0)),
        ],
        out_specs=pl.BlockSpec((1, ET, D), lambda m, t: (m, t, 0)),
        out_shape=jax.ShapeDtypeStruct((2, E, D), jnp.float32),
    )(eas, we, be)


def _gru_body(sx_ref, se_ref, h_ref, wm_ref, wzr_ref, uzr_ref, bzr_ref,
              wh_ref, uh_ref, bh_ref, o_ref):
    f32 = jnp.float32
    a = jnp.dot(sx_ref[0] + se_ref[0], wm_ref[0], preferred_element_type=f32, precision=lax.Precision.HIGHEST)
    h = h_ref[0]
    zr = jax.nn.sigmoid(
        jnp.dot(a, wzr_ref[0], preferred_element_type=f32, precision=lax.Precision.HIGHEST)
        + jnp.dot(h, uzr_ref[0], preferred_element_type=f32, precision=lax.Precision.HIGHEST) + bzr_ref[0])
    z = zr[:, :D]
    r = zr[:, D:]
    n = jnp.tanh(
        jnp.dot(a, wh_ref[0], preferred_element_type=f32, precision=lax.Precision.HIGHEST)
        + jnp.dot(r * h, uh_ref[0], preferred_element_type=f32, precision=lax.Precision.HIGHEST) + bh_ref[0])
    o_ref[0] = (1.0 - z) * n + z * h


def _gru(sx, se, h, wm, wzr, uzr, bzr, wh, uh, bh):
    return pl.pallas_call(
        _gru_body,
        grid=(2, NT),
        in_specs=[
            pl.BlockSpec((1, 128, D), lambda m, t: (m, t, 0)),
            pl.BlockSpec((1, 128, D), lambda m, t: (m, t, 0)),
            pl.BlockSpec((1, 128, D), lambda m, t: (m, t, 0)),
            pl.BlockSpec((1, D, D), lambda m, t: (m, 0, 0)),
            pl.BlockSpec((1, D, 2 * D), lambda m, t: (m, 0, 0)),
            pl.BlockSpec((1, D, 2 * D), lambda m, t: (m, 0, 0)),
            pl.BlockSpec((1, 1, 2 * D), lambda m, t: (m, 0, 0)),
            pl.BlockSpec((1, D, D), lambda m, t: (m, 0, 0)),
            pl.BlockSpec((1, D, D), lambda m, t: (m, 0, 0)),
            pl.BlockSpec((1, 1, D), lambda m, t: (m, 0, 0)),
        ],
        out_specs=pl.BlockSpec((1, 128, D), lambda m, t: (m, t, 0)),
        out_shape=jax.ShapeDtypeStruct((2, NP, D), jnp.float32),
    )(sx, se, h, wm, wzr, uzr, bzr, wh, uh, bh)


def _meta_body(b1r_ref, b2r_ref, c1_ref, c2_ref, s2i_ref, c2i_ref, acc1, acc2):
    t = pl.program_id(0)
    iota_b = lax.broadcasted_iota(jnp.int32, (NB, 128), 0).astype(jnp.float32)

    @pl.when(t == 0)
    def _():
        acc1[...] = jnp.zeros((NB, 1), jnp.float32)
        acc2[...] = jnp.zeros((NB, 1), jnp.float32)

    oh1 = (iota_b == b1r_ref[0]).astype(jnp.float32)
    oh2 = (iota_b == b2r_ref[0]).astype(jnp.float32)
    ones = jnp.ones((128, 1), jnp.float32)
    acc1[...] += jnp.dot(oh1, ones, preferred_element_type=jnp.float32, precision=lax.Precision.HIGHEST)
    acc2[...] += jnp.dot(oh2, ones, preferred_element_type=jnp.float32, precision=lax.Precision.HIGHEST)

    @pl.when(t == NT - 1)
    def _():
        c1_ref[...] = acc1[...]
        c2_ref[...] = acc2[...]
        lt = (lax.broadcasted_iota(jnp.int32, (NB, NB), 1)
              < lax.broadcasted_iota(jnp.int32, (NB, NB), 0)
              ).astype(jnp.float32)
        s2 = jnp.dot(lt, acc2[...], preferred_element_type=jnp.float32, precision=lax.Precision.HIGHEST)
        s2i_ref[...] = s2.astype(jnp.int32)
        c2i_ref[...] = acc2[...].astype(jnp.int32)


def _meta(b1r, b2r):
    return pl.pallas_call(
        _meta_body,
        grid=(NT,),
        in_specs=[
            pl.BlockSpec((1, 1, 128), lambda t: (t, 0, 0)),
            pl.BlockSpec((1, 1, 128), lambda t: (t, 0, 0)),
        ],
        out_specs=[
            pl.BlockSpec((NB, 1), lambda t: (0, 0)),
            pl.BlockSpec((NB, 1), lambda t: (0, 0)),
            pl.BlockSpec((NB, 1), lambda t: (0, 0)),
            pl.BlockSpec((NB, 1), lambda t: (0, 0)),
        ],
        out_shape=[
            jax.ShapeDtypeStruct((NB, 1), jnp.float32),
            jax.ShapeDtypeStruct((NB, 1), jnp.float32),
            jax.ShapeDtypeStruct((NB, 1), jnp.int32),
            jax.ShapeDtypeStruct((NB, 1), jnp.int32),
        ],
        scratch_shapes=[
            pltpu.VMEM((NB, 1), jnp.float32),
            pltpu.VMEM((NB, 1), jnp.float32),
        ],
    )(b1r, b2r)


def _eye128():
    return (lax.broadcasted_iota(jnp.int32, (128, 128), 0)
            == lax.broadcasted_iota(jnp.int32, (128, 128), 1)
            ).astype(jnp.float32)


def _pool_body(x1_ref, x2f_ref, b1r_ref, b2r_ref, b2f_ref,
               c1_ref, c2_ref, s2i_ref, c2i_ref, tb_ref,
               t1_ref, t2_ref, fus_ref, smax_acc):
    f32 = jnp.float32
    i = pl.program_id(0)

    @pl.when(i == 0)
    def _():
        t1_ref[...] = jnp.zeros((NB, D), f32)
        t2_ref[...] = jnp.zeros((NB, D), f32)
        smax_acc[...] = jnp.full((NB, 1), -BIGNEG, f32)

    x1t = x1_ref[0]
    b1row = b1r_ref[0]
    b1col = lax.dot_general(_eye128(), b1row, (((1,), (1,)), ((), ())),
                            preferred_element_type=f32, precision=lax.Precision.HIGHEST)
    iota_col = lax.broadcasted_iota(jnp.int32, (NB, 128), 0).astype(f32)
    oh1 = (iota_col == b1row).astype(f32)
    oh2 = (iota_col == b2r_ref[0]).astype(f32)
    t1_ref[...] += jnp.dot(oh1, x1t, preferred_element_type=f32, precision=lax.Precision.HIGHEST)
    t2_ref[...] += jnp.dot(oh2, x2f_ref[0, pl.ds(i * 128, 128), :],
                           preferred_element_type=f32, precision=lax.Precision.HIGHEST)

    bmin = jnp.minimum(tb_ref[i, 0], NB - 1)
    bmax = jnp.minimum(tb_ref[i, 1], NB - 1)
    cs = s2i_ref[bmin, 0]
    ce = s2i_ref[bmax, 0] + c2i_ref[bmax, 0]
    jlo = lax.div(cs, 128)
    jhi = lax.div(ce + 127, 128)

    def col_step(j, rmax):
        x2t = x2f_ref[0, pl.ds(j * 128, 128), :]
        s = lax.dot_general(x1t, x2t, (((1,), (1,)), ((), ())),
                            preferred_element_type=f32, precision=lax.Precision.HIGHEST)
        b2row = b2f_ref[pl.ds(j, 1), :]
        m = jnp.where(b1col == b2row, s, -BIGNEG)
        return jnp.maximum(rmax, jnp.max(m, axis=1, keepdims=True))

    rmax = lax.fori_loop(jlo, jhi, col_step,
                         jnp.full((128, 1), -BIGNEG, f32))
    rmax_row = lax.dot_general(rmax, _eye128(), (((0,), (0,)), ((), ())),
                               preferred_element_type=f32, precision=lax.Precision.HIGHEST)
    contrib = jnp.max(jnp.where(iota_col == b1row, rmax_row, -BIGNEG),
                      axis=1, keepdims=True)
    smax_acc[...] = jnp.maximum(smax_acc[...], contrib)

    @pl.when(i == NT - 1)
    def _():
        ssum = jnp.sum(t1_ref[...] * t2_ref[...], axis=1, keepdims=True)
        mean = ssum / (c1_ref[...] * c2_ref[...])
        fus_ref[...] = jnp.concatenate([smax_acc[...], mean], axis=1)


def _pool(x, b1r3, b2r3, b2r, c1f, c2f, s2i, c2i, tb):
    return pl.pallas_call(
        _pool_body,
        grid=(NT,),
        in_specs=[
            pl.BlockSpec((1, 128, D), lambda t: (0, t, 0)),
            pl.BlockSpec((1, NP, D), lambda t: (1, 0, 0)),
            pl.BlockSpec((1, 1, 128), lambda t: (t, 0, 0)),
            pl.BlockSpec((1, 1, 128), lambda t: (t, 0, 0)),
            pl.BlockSpec((NT, 128), lambda t: (0, 0)),
            pl.BlockSpec((NB, 1), lambda t: (0, 0)),
            pl.BlockSpec((NB, 1), lambda t: (0, 0)),
            pl.BlockSpec(memory_space=pltpu.SMEM),
            pl.BlockSpec(memory_space=pltpu.SMEM),
            pl.BlockSpec(memory_space=pltpu.SMEM),
        ],
        out_specs=[
            pl.BlockSpec((NB, D), lambda t: (0, 0)),
            pl.BlockSpec((NB, D), lambda t: (0, 0)),
            pl.BlockSpec((NB, 2), lambda t: (0, 0)),
        ],
        out_shape=[
            jax.ShapeDtypeStruct((NB, D), jnp.float32),
            jax.ShapeDtypeStruct((NB, D), jnp.float32),
            jax.ShapeDtypeStruct((NB, 2), jnp.float32),
        ],
        scratch_shapes=[pltpu.VMEM((NB, 1), jnp.float32)],
    )(x, x, b1r3, b2r3, b2r, c1f, c2f, s2i, c2i, tb)


def _head_body(x_ref, b1r_ref, b2r_ref, c1_ref, c2_ref, t1_ref, t2_ref,
               fus_ref, wf_ref, bf_ref, wo1_ref, bo1_ref, wo2_ref, bo2_ref,
               o_ref):
    f32 = jnp.float32
    iota_col = lax.broadcasted_iota(jnp.int32, (NB, 128), 0).astype(f32)
    eye = _eye128()
    os = []
    for m in range(2):
        br_ref = b1r_ref if m == 0 else b2r_ref
        cnt = c1_ref[...] if m == 0 else c2_ref[...]
        t = t1_ref[...] if m == 0 else t2_ref[...]

        def keyrow(tt):
            kc = x_ref[m, pl.ds(tt * 128, 128), 127:128]
            return lax.dot_general(kc, eye, (((0,), (0,)), ((), ())),
                                   preferred_element_type=f32, precision=lax.Precision.HIGHEST)

        def tile_scan(excl0, excl1):
            def step(tt, carry):
                v, ix = carry
                krow = keyrow(tt)
                oh = iota_col == br_ref[pl.ds(tt, 1), :]
                gcol = (lax.broadcasted_iota(jnp.int32, (NB, 128), 1).astype(f32)
                        + 128.0 * tt.astype(f32))
                mm = jnp.where(oh, jnp.broadcast_to(krow, (NB, 128)), NEG)
                if excl0 is not None:
                    mm = jnp.where(gcol == excl0, NEG, mm)
                if excl1 is not None:
                    mm = jnp.where(gcol == excl1, NEG, mm)
                tv = jnp.max(mm, axis=1, keepdims=True)
                tix = jnp.min(jnp.where(mm == tv, gcol, BIGF),
                              axis=1, keepdims=True)
                better = tv > v
                return (jnp.where(better, tv, v),
                        jnp.where(better, tix, ix))

            init = (jnp.full((NB, 1), NEG, f32), jnp.full((NB, 1), BIGF, f32))
            return lax.fori_loop(0, NT, step, init)

        _, i0 = tile_scan(None, None)
        _, i1 = tile_scan(i0, None)
        _, i2 = tile_scan(i0, i1)

        def gather(tt, accs):
            a0, a1, a2 = accs
            xt = x_ref[m, pl.ds(tt * 128, 128), :]
            gcol = (lax.broadcasted_iota(jnp.int32, (NB, 128), 1).astype(f32)
                    + 128.0 * tt.astype(f32))
            a0 += jnp.dot((gcol == i0).astype(f32), xt,
                          preferred_element_type=f32, precision=lax.Precision.HIGHEST)
            a1 += jnp.dot((gcol == i1).astype(f32), xt,
                          preferred_element_type=f32, precision=lax.Precision.HIGHEST)
            a2 += jnp.dot((gcol == i2).astype(f32), xt,
                          preferred_element_type=f32, precision=lax.Precision.HIGHEST)
            return a0, a1, a2

        z = jnp.zeros((NB, D), f32)
        a0, a1, a2 = lax.fori_loop(0, NT, gather, (z, z, z))
        a0 = jnp.where(cnt > 0.0, a0, 0.0)
        a1 = jnp.where(cnt > 1.0, a1, 0.0)
        a2 = jnp.where(cnt > 2.0, a2, 0.0)
        mean = t / jnp.maximum(cnt, 1.0)
        gp = jnp.concatenate([mean, t, a0, a1, a2], axis=1)
        os.append(jnp.dot(gp, wf_ref[m], preferred_element_type=f32, precision=lax.Precision.HIGHEST)
                  + bf_ref[m])
    cat = jnp.concatenate(
        [os[0], os[1], fus_ref[0], fus_ref[1], fus_ref[2]], axis=1)
    h1 = jnp.dot(cat, wo1_ref[...], preferred_element_type=f32, precision=lax.Precision.HIGHEST) + bo1_ref[...]
    o_ref[...] = (jnp.dot(h1, wo2_ref[...], preferred_element_type=f32, precision=lax.Precision.HIGHEST)
                  + bo2_ref[...])


def _head(x, b1r, b2r, c1f, c2f, t1, t2, fus, wf, bf, wo1, bo1, wo2, bo2):
    return pl.pallas_call(
        _head_body,
        in_specs=[
            pl.BlockSpec((2, NP, D), lambda: (0, 0, 0)),
            pl.BlockSpec((NT, 128), lambda: (0, 0)),
            pl.BlockSpec((NT, 128), lambda: (0, 0)),
            pl.BlockSpec((NB, 1), lambda: (0, 0)),
            pl.BlockSpec((NB, 1), lambda: (0, 0)),
            pl.BlockSpec((NB, D), lambda: (0, 0)),
            pl.BlockSpec((NB, D), lambda: (0, 0)),
            pl.BlockSpec((3, NB, 2), lambda: (0, 0, 0)),
            pl.BlockSpec((2, 5 * D, D), lambda: (0, 0, 0)),
            pl.BlockSpec((2, 1, D), lambda: (0, 0, 0)),
            pl.BlockSpec((2 * D + 6, D), lambda: (0, 0)),
            pl.BlockSpec((1, D), lambda: (0, 0)),
            pl.BlockSpec((D, 2), lambda: (0, 0)),
            pl.BlockSpec((1, 2), lambda: (0, 0)),
        ],
        out_specs=pl.BlockSpec((NB, 2), lambda: (0, 0)),
        out_shape=jax.ShapeDtypeStruct((NB, 2), jnp.float32),
    )(x, b1r, b2r, c1f, c2f, t1, t2, fus, wf, bf, wo1, bo1, wo2, bo2)


def kernel(mol1_x, mol1_edge_index, mol1_edge_attr, mol1_batch,
           mol2_x, mol2_edge_index, mol2_edge_attr, mol2_batch, params):
    p = params
    f32 = jnp.float32

    xs = jnp.zeros((2, NP, D), f32)
    xs = xs.at[0, :N].set(mol1_x).at[1, :N].set(mol2_x)
    eas = jnp.stack([mol1_edge_attr, mol2_edge_attr])
    b1p = jnp.concatenate([mol1_batch,
                           jnp.full((NP - N,), NB, jnp.int32)])
    b2p = jnp.concatenate([mol2_batch,
                           jnp.full((NP - N,), NB, jnp.int32)])
    b1r = b1p.reshape(NT, 128).astype(f32)
    b2r = b2p.reshape(NT, 128).astype(f32)
    b1r3 = b1r.reshape(NT, 1, 128)
    b2r3 = b2r.reshape(NT, 1, 128)
    tb = jnp.stack([b1p[::128], b1p[127::128]], axis=1)
    srcg = jnp.concatenate([mol1_edge_index[0],
                            mol2_edge_index[0] + NP])
    dstg = jnp.concatenate([mol1_edge_index[1],
                            mol2_edge_index[1]])
    zeros_np = jnp.zeros((NP, D), f32)

    def stack2(*names):
        return [jnp.stack([p['m1_' + nm], p['m2_' + nm]]) for nm in names]

    w0, we, wm, wh, uh = stack2('W0', 'We', 'Wm', 'Wh', 'Uh')
    b0, be, bh = [b.reshape(2, 1, D) for b in stack2('b0', 'be', 'bh')]
    wzr = jnp.concatenate(
        [jnp.stack([p['m1_Wz'], p['m2_Wz']]),
         jnp.stack([p['m1_Wr'], p['m2_Wr']])], axis=2)
    uzr = jnp.concatenate(
        [jnp.stack([p['m1_Uz'], p['m2_Uz']]),
         jnp.stack([p['m1_Ur'], p['m2_Ur']])], axis=2)
    bzr = jnp.concatenate(
        [jnp.stack([p['m1_bz'], p['m2_bz']]),
         jnp.stack([p['m1_br'], p['m2_br']])], axis=1).reshape(2, 1, 2 * D)
    wf = jnp.stack([p['m1_Wf'], p['m2_Wf']])
    bf = jnp.stack([p['m1_bf'], p['m2_bf']]).reshape(2, 1, D)

    x = _init_x(xs, w0, b0)
    y = _edge_feat(eas, we, be).reshape(2 * E, D)
    se = _segsum_linear(y, dstg, zeros_np).reshape(2, NP, D)
    c1f, c2f, s2i, c2i = _meta(b1r3, b2r3)

    fus = []
    t1 = t2 = None
    for _ in range(3):
        sx = _segsum_gather(x.reshape(2 * NP, D), srcg, dstg,
                            zeros_np).reshape(2, NP, D)
        x = _gru(sx, se, x, wm, wzr, uzr, bzr, wh, uh, bh)
        t1, t2, f = _pool(x, b1r3, b2r3, b2r, c1f, c2f, s2i, c2i, tb)
        fus.append(f)

    fus3 = jnp.stack(fus)
    return _head(x, b1r, b2r, c1f, c2f, t1, t2, fus3, wf, bf,
                 p['Wo1'], p['bo1'].reshape(1, D),
                 p['Wo2'], p['bo2'].reshape(1, 2))
